# baseline (speedup 1.0000x reference)
import jax
import jax.numpy as jnp
from jax.experimental import pallas as pl
from jax.experimental.pallas import tpu as pltpu

_BT = 16


def _conv_body(x1_r, w1_r, b1_r, w2_r, b2_r, w3_r, b3_r, fcw_r, fcb_r,
               feats_r, ycat_s, x2_s, ycat2_s, y2_s, ycat3_s, y3_s):
    bt = x1_r.shape[0]
    f32 = jnp.float32

    ycat_s[...] = jnp.dot(x1_r[...].reshape(bt * 441, 64), w1_r[...],
                          preferred_element_type=f32).reshape(bt, 21, 21, 128)
    acc = ycat_s[:, 0:20, 0:20, 0:32]
    acc = acc + ycat_s[:, 0:20, 1:21, 32:64]
    acc = acc + ycat_s[:, 1:21, 0:20, 64:96]
    acc = acc + ycat_s[:, 1:21, 1:21, 96:128]
    y1 = jnp.maximum(acc + b1_r[...].reshape(1, 1, 1, 32), 0.0)

    x2 = y1.reshape(bt, 10, 2, 10, 2, 32).transpose(0, 1, 3, 2, 4, 5)
    x2_s[...] = x2.reshape(bt, 10, 10, 128)

    ycat2_s[...] = jnp.dot(x2_s[...].reshape(bt * 100, 128), w2_r[...],
                           preferred_element_type=f32).reshape(bt, 10, 10, 256)
    acc2 = ycat2_s[:, 0:9, 0:9, 0:64]
    acc2 = acc2 + ycat2_s[:, 0:9, 1:10, 64:128]
    acc2 = acc2 + ycat2_s[:, 1:10, 0:9, 128:192]
    acc2 = acc2 + ycat2_s[:, 1:10, 1:10, 192:256]
    y2_s[...] = jnp.maximum(acc2 + b2_r[...].reshape(1, 1, 1, 64), 0.0)

    ycat3_s[...] = jnp.dot(y2_s[...].reshape(bt * 81, 64), w3_r[...],
                           preferred_element_type=f32).reshape(bt, 9, 9, 576)
    acc3 = ycat3_s[:, 0:7, 0:7, 0:64]
    for kh in range(3):
        for kw in range(3):
            if kh == 0 and kw == 0:
                continue
            k = 3 * kh + kw
            acc3 = acc3 + ycat3_s[:, kh:kh + 7, kw:kw + 7,
                                  64 * k:64 * k + 64]
    y3_s[...] = jnp.maximum(
        acc3 + b3_r[...].reshape(1, 1, 1, 64), 0.0).reshape(bt, 3136)

    feats = jnp.dot(y3_s[...], fcw_r[...], preferred_element_type=f32)
    feats_r[...] = jnp.maximum(feats + fcb_r[...], 0.0)


def _head_body(feats_r, hid_r, wih_r, whh_r, bih_r, bhh_r, rw_r, rb_r,
               ew1_r, eb1_r, ew2_r, eb2_r, sel_r, q_r, p_r, h_r):
    feats = feats_r[...]
    hid = hid_r[...]
    f32 = jnp.float32

    gi = jnp.dot(feats, wih_r[...], preferred_element_type=f32) + bih_r[...]
    gh = jnp.dot(hid, whh_r[...], preferred_element_type=f32) + bhh_r[...]
    r = jax.nn.sigmoid(gi[:, 0:128] + gh[:, 0:128])
    z = jax.nn.sigmoid(gi[:, 128:256] + gh[:, 128:256])
    n = jnp.tanh(gi[:, 256:384] + r * gh[:, 256:384])
    h_new = (1.0 - z) * n + z * hid

    logits = jnp.dot(h_new, rw_r[...], preferred_element_type=f32) + rb_r[...]
    m = jnp.max(logits, axis=-1, keepdims=True)
    e = jnp.exp(logits - m)
    p = e / jnp.sum(e, axis=-1, keepdims=True)

    h1 = jnp.dot(feats, ew1_r[...], preferred_element_type=f32) + eb1_r[...]
    h1 = jnp.maximum(h1, 0.0)
    pe = jnp.dot(p, sel_r[...], preferred_element_type=f32)
    q = jnp.dot(h1 * pe, ew2_r[...], preferred_element_type=f32)
    q = q + jnp.dot(p, eb2_r[...], preferred_element_type=f32)

    q_r[...] = q
    p_r[...] = p
    h_r[...] = h_new


def kernel(obs, hidden, conv1_w, conv1_b, conv2_w, conv2_b, conv3_w,
           conv3_b, fc_w, fc_b, gru_w_ih, gru_w_hh, gru_b_ih, gru_b_hh,
           rout_w, rout_b, exp_w1, exp_b1, exp_w2, exp_b2):
    B = obs.shape[0]
    bt = _BT
    assert B % bt == 0

    x1 = obs.reshape(B, 4, 21, 4, 21, 4).transpose(0, 2, 4, 3, 5, 1)
    x1 = x1.reshape(B, 21, 21, 64)
    w1 = conv1_w.reshape(32, 4, 2, 4, 2, 4).transpose(2, 4, 3, 5, 1, 0)
    w1 = w1.reshape(4, 64, 32).transpose(1, 0, 2).reshape(64, 128)
    w2 = conv2_w.reshape(64, 32, 2, 2, 2, 2).transpose(2, 4, 3, 5, 1, 0)
    w2 = w2.reshape(4, 128, 64).transpose(1, 0, 2).reshape(128, 256)
    w3 = conv3_w.transpose(2, 3, 1, 0).reshape(9, 64, 64)
    w3 = w3.transpose(1, 0, 2).reshape(64, 576)
    fcw = fc_w.reshape(64, 7, 7, 512).transpose(1, 2, 0, 3).reshape(3136, 512)
    ew1 = exp_w1.transpose(1, 0, 2).reshape(512, 2048)
    eb1 = exp_b1.reshape(1, 2048)
    ew2 = exp_w2.reshape(2048, 18)
    sel = jnp.repeat(jnp.eye(8, dtype=jnp.float32), 256, axis=1)

    wspec2 = lambda a, b: pl.BlockSpec((a, b), lambda i: (0, 0))

    feats = pl.pallas_call(
        _conv_body,
        grid=(B // bt,),
        in_specs=[
            pl.BlockSpec((bt, 21, 21, 64), lambda i: (i, 0, 0, 0)),
            wspec2(64, 128),
            wspec2(1, 32),
            wspec2(128, 256),
            wspec2(1, 64),
            wspec2(64, 576),
            wspec2(1, 64),
            wspec2(3136, 512),
            wspec2(1, 512),
        ],
        out_specs=pl.BlockSpec((bt, 512), lambda i: (i, 0)),
        out_shape=jax.ShapeDtypeStruct((B, 512), jnp.float32),
        scratch_shapes=[
            pltpu.VMEM((bt, 21, 21, 128), jnp.float32),
            pltpu.VMEM((bt, 10, 10, 128), jnp.float32),
            pltpu.VMEM((bt, 10, 10, 256), jnp.float32),
            pltpu.VMEM((bt, 9, 9, 64), jnp.float32),
            pltpu.VMEM((bt, 9, 9, 576), jnp.float32),
            pltpu.VMEM((bt, 3136), jnp.float32),
        ],
        compiler_params=pltpu.CompilerParams(
            vmem_limit_bytes=60 * 1024 * 1024),
    )(x1, w1, conv1_b.reshape(1, 32), w2, conv2_b.reshape(1, 64),
      w3, conv3_b.reshape(1, 64), fcw, fc_b.reshape(1, 512))

    q, p, h = pl.pallas_call(
        _head_body,
        grid=(1,),
        in_specs=[
            pl.BlockSpec((B, 512), lambda i: (0, 0)),
            pl.BlockSpec((B, 128), lambda i: (0, 0)),
            wspec2(512, 384),
            wspec2(128, 384),
            wspec2(1, 384),
            wspec2(1, 384),
            wspec2(128, 8),
            wspec2(1, 8),
            wspec2(512, 2048),
            wspec2(1, 2048),
            wspec2(2048, 18),
            wspec2(8, 18),
            wspec2(8, 2048),
        ],
        out_specs=(
            pl.BlockSpec((B, 18), lambda i: (0, 0)),
            pl.BlockSpec((B, 8), lambda i: (0, 0)),
            pl.BlockSpec((B, 128), lambda i: (0, 0)),
        ),
        out_shape=(
            jax.ShapeDtypeStruct((B, 18), jnp.float32),
            jax.ShapeDtypeStruct((B, 8), jnp.float32),
            jax.ShapeDtypeStruct((B, 128), jnp.float32),
        ),
        compiler_params=pltpu.CompilerParams(
            vmem_limit_bytes=60 * 1024 * 1024),
    )(feats, hidden, gru_w_ih, gru_w_hh, gru_b_ih.reshape(1, 384),
      gru_b_hh.reshape(1, 384), rout_w, rout_b.reshape(1, 8), ew1, eb1,
      ew2, exp_b2, sel)
    return (q, p, h)


# speedup vs baseline: 1.0534x; 1.0534x over previous
import jax
import jax.numpy as jnp
from jax.experimental import pallas as pl
from jax.experimental.pallas import tpu as pltpu

_BT = 16


def _conv_body(x1_r, w1_r, b1_r, w2_r, b2_r, w3_r, b3_r, fcw_r, fcb_r,
               feats_r, ycat_s, x2_s, ycat2_s, y2_s, ycat3_s, y3_s):
    bt = x1_r.shape[0]
    f32 = jnp.float32

    ycat_s[...] = jnp.dot(x1_r[...].reshape(bt * 441, 64), w1_r[...],
                          preferred_element_type=f32).reshape(bt, 21, 21, 128)
    acc = ycat_s[:, 0:20, 0:20, 0:32]
    acc = acc + ycat_s[:, 0:20, 1:21, 32:64]
    acc = acc + ycat_s[:, 1:21, 0:20, 64:96]
    acc = acc + ycat_s[:, 1:21, 1:21, 96:128]
    y1 = jnp.maximum(acc + b1_r[...].reshape(1, 1, 1, 32), 0.0)
    y1 = y1.astype(jnp.bfloat16)

    x2 = y1.reshape(bt, 10, 2, 10, 2, 32).transpose(0, 1, 3, 2, 4, 5)
    x2_s[...] = x2.reshape(bt, 10, 10, 128)

    ycat2_s[...] = jnp.dot(x2_s[...].reshape(bt * 100, 128), w2_r[...],
                           preferred_element_type=f32).reshape(bt, 10, 10, 256)
    acc2 = ycat2_s[:, 0:9, 0:9, 0:64]
    acc2 = acc2 + ycat2_s[:, 0:9, 1:10, 64:128]
    acc2 = acc2 + ycat2_s[:, 1:10, 0:9, 128:192]
    acc2 = acc2 + ycat2_s[:, 1:10, 1:10, 192:256]
    y2_s[...] = jnp.maximum(
        acc2 + b2_r[...].reshape(1, 1, 1, 64), 0.0).astype(jnp.bfloat16)

    ycat3_s[...] = jnp.dot(y2_s[...].reshape(bt * 81, 64), w3_r[...],
                           preferred_element_type=f32).reshape(bt, 9, 9, 576)
    acc3 = ycat3_s[:, 0:7, 0:7, 0:64]
    for kh in range(3):
        for kw in range(3):
            if kh == 0 and kw == 0:
                continue
            k = 3 * kh + kw
            acc3 = acc3 + ycat3_s[:, kh:kh + 7, kw:kw + 7,
                                  64 * k:64 * k + 64]
    y3_s[...] = jnp.maximum(
        acc3 + b3_r[...].reshape(1, 1, 1, 64),
        0.0).astype(jnp.bfloat16).reshape(bt, 3136)

    feats = jnp.dot(y3_s[...], fcw_r[...], preferred_element_type=f32)
    feats_r[...] = jnp.maximum(feats + fcb_r[...], 0.0)


def _head_body(feats_r, hid_r, wih_r, whh_r, bih_r, bhh_r, rw_r, rb_r,
               ew1_r, eb1_r, ew2_r, eb2_r, sel_r, q_r, p_r, h_r):
    feats = feats_r[...]
    featsb = feats.astype(jnp.bfloat16)
    hid = hid_r[...]
    f32 = jnp.float32

    gi = jnp.dot(featsb, wih_r[...], preferred_element_type=f32) + bih_r[...]
    gh = jnp.dot(hid.astype(jnp.bfloat16), whh_r[...],
                 preferred_element_type=f32) + bhh_r[...]
    r = jax.nn.sigmoid(gi[:, 0:128] + gh[:, 0:128])
    z = jax.nn.sigmoid(gi[:, 128:256] + gh[:, 128:256])
    n = jnp.tanh(gi[:, 256:384] + r * gh[:, 256:384])
    h_new = (1.0 - z) * n + z * hid

    logits = jnp.dot(h_new, rw_r[...], preferred_element_type=f32) + rb_r[...]
    m = jnp.max(logits, axis=-1, keepdims=True)
    e = jnp.exp(logits - m)
    p = e / jnp.sum(e, axis=-1, keepdims=True)

    h1 = jnp.dot(featsb, ew1_r[...], preferred_element_type=f32) + eb1_r[...]
    h1 = jnp.maximum(h1, 0.0)
    pe = jnp.dot(p, sel_r[...], preferred_element_type=f32)
    q = jnp.dot((h1 * pe).astype(jnp.bfloat16), ew2_r[...],
                preferred_element_type=f32)
    q = q + jnp.dot(p, eb2_r[...], preferred_element_type=f32)

    q_r[...] = q
    p_r[...] = p
    h_r[...] = h_new


def kernel(obs, hidden, conv1_w, conv1_b, conv2_w, conv2_b, conv3_w,
           conv3_b, fc_w, fc_b, gru_w_ih, gru_w_hh, gru_b_ih, gru_b_hh,
           rout_w, rout_b, exp_w1, exp_b1, exp_w2, exp_b2):
    B = obs.shape[0]
    bt = _BT
    assert B % bt == 0

    x1 = obs.astype(jnp.bfloat16)
    x1 = x1.reshape(B, 4, 21, 4, 21, 4).transpose(0, 2, 4, 3, 5, 1)
    x1 = x1.reshape(B, 21, 21, 64)
    w1 = conv1_w.reshape(32, 4, 2, 4, 2, 4).transpose(2, 4, 3, 5, 1, 0)
    w1 = w1.reshape(4, 64, 32).transpose(1, 0, 2).reshape(64, 128)
    w1 = w1.astype(jnp.bfloat16)
    w2 = conv2_w.reshape(64, 32, 2, 2, 2, 2).transpose(2, 4, 3, 5, 1, 0)
    w2 = w2.reshape(4, 128, 64).transpose(1, 0, 2).reshape(128, 256)
    w2 = w2.astype(jnp.bfloat16)
    w3 = conv3_w.transpose(2, 3, 1, 0).reshape(9, 64, 64)
    w3 = w3.transpose(1, 0, 2).reshape(64, 576)
    w3 = w3.astype(jnp.bfloat16)
    fcw = fc_w.reshape(64, 7, 7, 512).transpose(1, 2, 0, 3).reshape(3136, 512)
    fcw = fcw.astype(jnp.bfloat16)
    ew1 = exp_w1.transpose(1, 0, 2).reshape(512, 2048).astype(jnp.bfloat16)
    eb1 = exp_b1.reshape(1, 2048)
    ew2 = exp_w2.reshape(2048, 18).astype(jnp.bfloat16)
    sel = jnp.repeat(jnp.eye(8, dtype=jnp.float32), 256, axis=1)

    wspec2 = lambda a, b: pl.BlockSpec((a, b), lambda i: (0, 0))

    feats = pl.pallas_call(
        _conv_body,
        grid=(B // bt,),
        in_specs=[
            pl.BlockSpec((bt, 21, 21, 64), lambda i: (i, 0, 0, 0)),
            wspec2(64, 128),
            wspec2(1, 32),
            wspec2(128, 256),
            wspec2(1, 64),
            wspec2(64, 576),
            wspec2(1, 64),
            wspec2(3136, 512),
            wspec2(1, 512),
        ],
        out_specs=pl.BlockSpec((bt, 512), lambda i: (i, 0)),
        out_shape=jax.ShapeDtypeStruct((B, 512), jnp.float32),
        scratch_shapes=[
            pltpu.VMEM((bt, 21, 21, 128), jnp.float32),
            pltpu.VMEM((bt, 10, 10, 128), jnp.bfloat16),
            pltpu.VMEM((bt, 10, 10, 256), jnp.float32),
            pltpu.VMEM((bt, 9, 9, 64), jnp.bfloat16),
            pltpu.VMEM((bt, 9, 9, 576), jnp.float32),
            pltpu.VMEM((bt, 3136), jnp.bfloat16),
        ],
        compiler_params=pltpu.CompilerParams(
            vmem_limit_bytes=60 * 1024 * 1024),
    )(x1, w1, conv1_b.reshape(1, 32), w2, conv2_b.reshape(1, 64),
      w3, conv3_b.reshape(1, 64), fcw, fc_b.reshape(1, 512))

    q, p, h = pl.pallas_call(
        _head_body,
        grid=(1,),
        in_specs=[
            pl.BlockSpec((B, 512), lambda i: (0, 0)),
            pl.BlockSpec((B, 128), lambda i: (0, 0)),
            wspec2(512, 384),
            wspec2(128, 384),
            wspec2(1, 384),
            wspec2(1, 384),
            wspec2(128, 8),
            wspec2(1, 8),
            wspec2(512, 2048),
            wspec2(1, 2048),
            wspec2(2048, 18),
            wspec2(8, 18),
            wspec2(8, 2048),
        ],
        out_specs=(
            pl.BlockSpec((B, 18), lambda i: (0, 0)),
            pl.BlockSpec((B, 8), lambda i: (0, 0)),
            pl.BlockSpec((B, 128), lambda i: (0, 0)),
        ),
        out_shape=(
            jax.ShapeDtypeStruct((B, 18), jnp.float32),
            jax.ShapeDtypeStruct((B, 8), jnp.float32),
            jax.ShapeDtypeStruct((B, 128), jnp.float32),
        ),
        compiler_params=pltpu.CompilerParams(
            vmem_limit_bytes=60 * 1024 * 1024),
    )(feats, hidden, gru_w_ih.astype(jnp.bfloat16),
      gru_w_hh.astype(jnp.bfloat16), gru_b_ih.reshape(1, 384),
      gru_b_hh.reshape(1, 384), rout_w, rout_b.reshape(1, 8), ew1, eb1,
      ew2, exp_b2, sel)
    return (q, p, h)


# speedup vs baseline: 1.3433x; 1.2752x over previous
import jax
import jax.numpy as jnp
from jax.experimental import pallas as pl
from jax.experimental.pallas import tpu as pltpu

_BT1 = 32
_BT2 = 32
bf16 = jnp.bfloat16


def _conv1_body(x1_r, w1_r, b1_r, y1_r):
    bt = x1_r.shape[0]
    f32 = jnp.float32
    R1 = bt * 512
    x1f = x1_r[...].reshape(R1, 64)
    M = R1 - 32
    m24 = M + 24
    xc1 = jnp.concatenate([x1f[0:m24], x1f[1:m24 + 1]], axis=1)
    acc1 = jnp.dot(xc1, w1_r[...], preferred_element_type=f32)
    y1 = jnp.maximum(acc1[0:M, 0:32] + acc1[24:M + 24, 32:64] + b1_r[...],
                     0.0)
    y1_r[0:M, :] = y1.astype(bf16)


def _conv23_body(x2_r, w2_r, b2_r, w3_r, b3_r, y3_r, y2_s):
    bt = x2_r.shape[0]
    f32 = jnp.float32
    R2 = bt * 192
    x2f = x2_r[...].reshape(R2, 128)
    M2 = R2 - 32
    m2b = M2 + 16
    xc2 = jnp.concatenate([x2f[0:m2b], x2f[1:m2b + 1]], axis=1)
    acc2 = jnp.dot(xc2, w2_r[...], preferred_element_type=f32)
    y2 = jnp.maximum(
        acc2[0:M2, 0:64] + acc2[16:M2 + 16, 64:128] + b2_r[...], 0.0)
    y2_s[0:M2, :] = y2.astype(bf16)

    y2f = y2_s[...]
    M3 = R2 - 48
    m3b = M3 + 32
    xc3 = jnp.concatenate([y2f[0:m3b], y2f[1:m3b + 1], y2f[2:m3b + 2]],
                          axis=1)
    acc3 = jnp.dot(xc3, w3_r[...], preferred_element_type=f32)
    y3 = jnp.maximum(acc3[0:M3, 0:64] + acc3[16:M3 + 16, 64:128]
                     + acc3[32:M3 + 32, 128:192] + b3_r[...], 0.0)
    y3_r[0:M3, :] = y3.astype(bf16)


def _head_body(y3_r, hid_r, fcw_r, fcb_r, wih_r, whh_r, bih_r, bhh_r,
               rw_r, rb_r, ew1_r, eb1_r, ew2_r, eb2_r, sel_r,
               q_r, p_r, h_r):
    f32 = jnp.float32
    hid = hid_r[...]

    feats = jnp.dot(y3_r[...], fcw_r[...], preferred_element_type=f32)
    feats = jnp.maximum(feats + fcb_r[...], 0.0)
    featsb = feats.astype(bf16)

    gi = jnp.dot(featsb, wih_r[...], preferred_element_type=f32) + bih_r[...]
    gh = jnp.dot(hid.astype(bf16), whh_r[...],
                 preferred_element_type=f32) + bhh_r[...]
    r = jax.nn.sigmoid(gi[:, 0:128] + gh[:, 0:128])
    z = jax.nn.sigmoid(gi[:, 128:256] + gh[:, 128:256])
    n = jnp.tanh(gi[:, 256:384] + r * gh[:, 256:384])
    h_new = (1.0 - z) * n + z * hid

    logits = jnp.dot(h_new, rw_r[...], preferred_element_type=f32) + rb_r[...]
    m = jnp.max(logits, axis=-1, keepdims=True)
    e = jnp.exp(logits - m)
    p = e / jnp.sum(e, axis=-1, keepdims=True)

    h1 = jnp.dot(featsb, ew1_r[...], preferred_element_type=f32) + eb1_r[...]
    h1 = jnp.maximum(h1, 0.0)
    pe = jnp.dot(p, sel_r[...], preferred_element_type=f32)
    q = jnp.dot((h1 * pe).astype(bf16), ew2_r[...],
                preferred_element_type=f32)
    q = q + jnp.dot(p, eb2_r[...], preferred_element_type=f32)

    q_r[...] = q
    p_r[...] = p
    h_r[...] = h_new


def kernel(obs, hidden, conv1_w, conv1_b, conv2_w, conv2_b, conv3_w,
           conv3_b, fc_w, fc_b, gru_w_ih, gru_w_hh, gru_b_ih, gru_b_hh,
           rout_w, rout_b, exp_w1, exp_b1, exp_w2, exp_b2):
    B = obs.shape[0]
    assert B % _BT1 == 0 and B % _BT2 == 0

    x1 = obs.astype(bf16).reshape(B, 4, 21, 4, 21, 4)
    x1 = x1.transpose(0, 2, 4, 3, 1, 5).reshape(B, 21, 21, 64)
    x1 = jnp.pad(x1, ((0, 0), (0, 0), (0, 3), (0, 0)))
    x1 = jnp.pad(x1.reshape(B, 504, 64), ((0, 0), (0, 8), (0, 0)))
    w1 = conv1_w.reshape(32, 4, 2, 4, 2, 4).transpose(4, 3, 1, 5, 2, 0)
    w1 = w1.reshape(128, 64).astype(bf16)
    w2 = conv2_w.reshape(64, 32, 2, 2, 2, 2).transpose(4, 3, 5, 1, 2, 0)
    w2 = w2.reshape(256, 128).astype(bf16)
    w3 = conv3_w.transpose(3, 1, 2, 0).reshape(192, 192).astype(bf16)
    fcw = fc_w.reshape(64, 7, 7, 512).transpose(1, 2, 0, 3)
    fcw = jnp.pad(fcw, ((0, 0), (0, 1), (0, 0), (0, 0)))
    fcw = fcw.reshape(3584, 512).astype(bf16)
    ew1 = exp_w1.transpose(1, 0, 2).reshape(512, 2048).astype(bf16)
    eb1 = exp_b1.reshape(1, 2048)
    ew2 = exp_w2.reshape(2048, 18).astype(bf16)
    sel = jnp.repeat(jnp.eye(8, dtype=jnp.float32), 256, axis=1)

    wspec2 = lambda a, b: pl.BlockSpec((a, b), lambda i: (0, 0))

    y1 = pl.pallas_call(
        _conv1_body,
        grid=(B // _BT1,),
        in_specs=[
            pl.BlockSpec((_BT1, 512, 64), lambda i: (i, 0, 0)),
            wspec2(128, 64),
            wspec2(1, 32),
        ],
        out_specs=pl.BlockSpec((_BT1 * 512, 32), lambda i: (i, 0)),
        out_shape=jax.ShapeDtypeStruct((B * 512, 32), bf16),
        compiler_params=pltpu.CompilerParams(
            vmem_limit_bytes=60 * 1024 * 1024),
    )(x1, w1, conv1_b.reshape(1, 32))

    y1v = y1.reshape(B, 512, 32)[:, 0:504, :].reshape(B, 21, 24, 32)
    y1v = y1v[:, 0:20, 0:20, :].reshape(B, 10, 2, 10, 2, 32)
    x2 = y1v.transpose(0, 1, 3, 2, 4, 5).reshape(B, 10, 10, 128)
    x2 = jnp.pad(x2, ((0, 0), (0, 2), (0, 6), (0, 0)))
    x2 = x2.reshape(B, 192, 128)

    y3 = pl.pallas_call(
        _conv23_body,
        grid=(B // _BT2,),
        in_specs=[
            pl.BlockSpec((_BT2, 192, 128), lambda i: (i, 0, 0)),
            wspec2(256, 128),
            wspec2(1, 64),
            wspec2(192, 192),
            wspec2(1, 64),
        ],
        out_specs=pl.BlockSpec((_BT2 * 192, 64), lambda i: (i, 0)),
        out_shape=jax.ShapeDtypeStruct((B * 192, 64), bf16),
        scratch_shapes=[pltpu.VMEM((_BT2 * 192, 64), bf16)],
        compiler_params=pltpu.CompilerParams(
            vmem_limit_bytes=60 * 1024 * 1024),
    )(x2, w2, conv2_b.reshape(1, 64), w3, conv3_b.reshape(1, 64))

    y3q = y3.reshape(B, 12, 16, 64)[:, 0:7, 0:8, :].reshape(B, 3584)

    q, p, h = pl.pallas_call(
        _head_body,
        grid=(1,),
        in_specs=[
            pl.BlockSpec((B, 3584), lambda i: (0, 0)),
            pl.BlockSpec((B, 128), lambda i: (0, 0)),
            wspec2(3584, 512),
            wspec2(1, 512),
            wspec2(512, 384),
            wspec2(128, 384),
            wspec2(1, 384),
            wspec2(1, 384),
            wspec2(128, 8),
            wspec2(1, 8),
            wspec2(512, 2048),
            wspec2(1, 2048),
            wspec2(2048, 18),
            wspec2(8, 18),
            wspec2(8, 2048),
        ],
        out_specs=(
            pl.BlockSpec((B, 18), lambda i: (0, 0)),
            pl.BlockSpec((B, 8), lambda i: (0, 0)),
            pl.BlockSpec((B, 128), lambda i: (0, 0)),
        ),
        out_shape=(
            jax.ShapeDtypeStruct((B, 18), jnp.float32),
            jax.ShapeDtypeStruct((B, 8), jnp.float32),
            jax.ShapeDtypeStruct((B, 128), jnp.float32),
        ),
        compiler_params=pltpu.CompilerParams(
            vmem_limit_bytes=60 * 1024 * 1024),
    )(y3q, hidden, fcw, fc_b.reshape(1, 512), gru_w_ih.astype(bf16),
      gru_w_hh.astype(bf16), gru_b_ih.reshape(1, 384),
      gru_b_hh.reshape(1, 384), rout_w, rout_b.reshape(1, 8), ew1, eb1,
      ew2, exp_b2, sel)
    return (q, p, h)


# speedup vs baseline: 1.5033x; 1.1191x over previous
import jax
import jax.numpy as jnp
from jax.experimental import pallas as pl
from jax.experimental.pallas import tpu as pltpu

_BT1 = 32
_BT2 = 32
bf16 = jnp.bfloat16


def _conv1_body(x1_r, w1_r, b1_r, y1_r):
    bt = x1_r.shape[0]
    f32 = jnp.float32
    R1 = bt * 528
    x1f = x1_r[...].reshape(R1, 64)
    M = R1 - 32
    m24 = M + 24
    xc1 = jnp.concatenate([x1f[0:m24], x1f[1:m24 + 1]], axis=1)
    acc1 = jnp.dot(xc1, w1_r[...], preferred_element_type=f32)
    y1 = jnp.maximum(acc1[0:M, 0:32] + acc1[24:M + 24, 32:64] + b1_r[...],
                     0.0)
    y1_r[0:M, :] = y1.astype(bf16)


def _conv23_body(x2_r, w2_r, b2_r, w3_r, b3_r, y3_r, y2_s, y3_s):
    bt = x2_r.shape[0]
    f32 = jnp.float32
    R2 = bt * 192
    x2f = x2_r[...].reshape(R2, 128)
    M2 = R2 - 32
    m2b = M2 + 16
    xc2 = jnp.concatenate([x2f[0:m2b], x2f[1:m2b + 1]], axis=1)
    acc2 = jnp.dot(xc2, w2_r[...], preferred_element_type=f32)
    y2 = jnp.maximum(
        acc2[0:M2, 0:64] + acc2[16:M2 + 16, 64:128] + b2_r[...], 0.0)
    y2_s[0:M2, :] = y2.astype(bf16)

    y2f = y2_s[...]
    M3 = R2 - 48
    m3b = M3 + 32
    xc3 = jnp.concatenate([y2f[0:m3b], y2f[1:m3b + 1], y2f[2:m3b + 2]],
                          axis=1)
    acc3 = jnp.dot(xc3, w3_r[...], preferred_element_type=f32)
    y3 = jnp.maximum(acc3[0:M3, 0:64] + acc3[16:M3 + 16, 64:128]
                     + acc3[32:M3 + 32, 128:192] + b3_r[...], 0.0)
    y3_s[0:M3, :] = y3.astype(bf16)
    y3_r[...] = y3_s[...].reshape(bt, 12, 16, 64)[:, 0:7, 0:8, :]


def _head_body(y3_r, hid_r, fcw_r, fcb_r, wih_r, whh_r, bih_r, bhh_r,
               rw_r, rb_r, ew1_r, eb1_r, ew2_r, eb2_r, sel_r,
               q_r, p_r, h_r):
    f32 = jnp.float32
    hid = hid_r[...]

    feats = jnp.dot(y3_r[...], fcw_r[...], preferred_element_type=f32)
    feats = jnp.maximum(feats + fcb_r[...], 0.0)
    featsb = feats.astype(bf16)

    gi = jnp.dot(featsb, wih_r[...], preferred_element_type=f32) + bih_r[...]
    gh = jnp.dot(hid.astype(bf16), whh_r[...],
                 preferred_element_type=f32) + bhh_r[...]
    r = jax.nn.sigmoid(gi[:, 0:128] + gh[:, 0:128])
    z = jax.nn.sigmoid(gi[:, 128:256] + gh[:, 128:256])
    n = jnp.tanh(gi[:, 256:384] + r * gh[:, 256:384])
    h_new = (1.0 - z) * n + z * hid

    logits = jnp.dot(h_new, rw_r[...], preferred_element_type=f32) + rb_r[...]
    m = jnp.max(logits, axis=-1, keepdims=True)
    e = jnp.exp(logits - m)
    p = e / jnp.sum(e, axis=-1, keepdims=True)

    h1 = jnp.dot(featsb, ew1_r[...], preferred_element_type=f32) + eb1_r[...]
    h1 = jnp.maximum(h1, 0.0)
    pe = jnp.dot(p, sel_r[...], preferred_element_type=f32)
    q = jnp.dot((h1 * pe).astype(bf16), ew2_r[...],
                preferred_element_type=f32)
    q = q + jnp.dot(p, eb2_r[...], preferred_element_type=f32)

    q_r[...] = q
    p_r[...] = p
    h_r[...] = h_new


def kernel(obs, hidden, conv1_w, conv1_b, conv2_w, conv2_b, conv3_w,
           conv3_b, fc_w, fc_b, gru_w_ih, gru_w_hh, gru_b_ih, gru_b_hh,
           rout_w, rout_b, exp_w1, exp_b1, exp_w2, exp_b2):
    B = obs.shape[0]
    assert B % _BT1 == 0 and B % _BT2 == 0

    xt = obs.astype(bf16).reshape(B, 4, 21, 4, 21, 4)
    xt = xt.transpose(0, 2, 4, 3, 1, 5).reshape(B, 21, 21, 64)
    x1 = jnp.zeros((B, 22, 24, 64), bf16).at[:, 0:21, 0:21, :].set(xt)
    x1 = x1.reshape(B, 528, 64)
    w1 = conv1_w.reshape(32, 4, 2, 4, 2, 4).transpose(4, 3, 1, 5, 2, 0)
    w1 = w1.reshape(128, 64).astype(bf16)
    w2 = conv2_w.reshape(64, 32, 2, 2, 2, 2).transpose(4, 3, 5, 1, 2, 0)
    w2 = w2.reshape(256, 128).astype(bf16)
    w3 = conv3_w.transpose(3, 1, 2, 0).reshape(192, 192).astype(bf16)
    fcw = fc_w.reshape(64, 7, 7, 512).transpose(1, 2, 0, 3)
    fcw = jnp.pad(fcw, ((0, 0), (0, 1), (0, 0), (0, 0)))
    fcw = fcw.reshape(3584, 512).astype(bf16)
    ew1 = exp_w1.transpose(1, 0, 2).reshape(512, 2048).astype(bf16)
    eb1 = exp_b1.reshape(1, 2048)
    ew2 = exp_w2.reshape(2048, 18).astype(bf16)
    sel = jnp.repeat(jnp.eye(8, dtype=jnp.float32), 256, axis=1)

    wspec2 = lambda a, b: pl.BlockSpec((a, b), lambda i: (0, 0))

    y1 = pl.pallas_call(
        _conv1_body,
        grid=(B // _BT1,),
        in_specs=[
            pl.BlockSpec((_BT1, 528, 64), lambda i: (i, 0, 0)),
            wspec2(128, 64),
            wspec2(1, 32),
        ],
        out_specs=pl.BlockSpec((_BT1 * 528, 32), lambda i: (i, 0)),
        out_shape=jax.ShapeDtypeStruct((B * 528, 32), bf16),
        compiler_params=pltpu.CompilerParams(
            vmem_limit_bytes=60 * 1024 * 1024),
    )(x1, w1, conv1_b.reshape(1, 32))

    y1v = y1.reshape(B, 22, 24, 32)[:, 0:20, 0:20, :]
    y1v = y1v.reshape(B, 10, 2, 10, 2, 32).transpose(0, 1, 3, 2, 4, 5)
    x2 = jnp.zeros((B, 12, 16, 128), bf16).at[:, 0:10, 0:10, :].set(
        y1v.reshape(B, 10, 10, 128))
    x2 = x2.reshape(B, 192, 128)

    y3 = pl.pallas_call(
        _conv23_body,
        grid=(B // _BT2,),
        in_specs=[
            pl.BlockSpec((_BT2, 192, 128), lambda i: (i, 0, 0)),
            wspec2(256, 128),
            wspec2(1, 64),
            wspec2(192, 192),
            wspec2(1, 64),
        ],
        out_specs=pl.BlockSpec((_BT2, 7, 8, 64), lambda i: (i, 0, 0, 0)),
        out_shape=jax.ShapeDtypeStruct((B, 7, 8, 64), bf16),
        scratch_shapes=[pltpu.VMEM((_BT2 * 192, 64), bf16),
                        pltpu.VMEM((_BT2 * 192, 64), bf16)],
        compiler_params=pltpu.CompilerParams(
            vmem_limit_bytes=60 * 1024 * 1024),
    )(x2, w2, conv2_b.reshape(1, 64), w3, conv3_b.reshape(1, 64))

    y3q = y3.reshape(B, 3584)

    q, p, h = pl.pallas_call(
        _head_body,
        grid=(1,),
        in_specs=[
            pl.BlockSpec((B, 3584), lambda i: (0, 0)),
            pl.BlockSpec((B, 128), lambda i: (0, 0)),
            wspec2(3584, 512),
            wspec2(1, 512),
            wspec2(512, 384),
            wspec2(128, 384),
            wspec2(1, 384),
            wspec2(1, 384),
            wspec2(128, 8),
            wspec2(1, 8),
            wspec2(512, 2048),
            wspec2(1, 2048),
            wspec2(2048, 18),
            wspec2(8, 18),
            wspec2(8, 2048),
        ],
        out_specs=(
            pl.BlockSpec((B, 18), lambda i: (0, 0)),
            pl.BlockSpec((B, 8), lambda i: (0, 0)),
            pl.BlockSpec((B, 128), lambda i: (0, 0)),
        ),
        out_shape=(
            jax.ShapeDtypeStruct((B, 18), jnp.float32),
            jax.ShapeDtypeStruct((B, 8), jnp.float32),
            jax.ShapeDtypeStruct((B, 128), jnp.float32),
        ),
        compiler_params=pltpu.CompilerParams(
            vmem_limit_bytes=60 * 1024 * 1024),
    )(y3q, hidden, fcw, fc_b.reshape(1, 512), gru_w_ih.astype(bf16),
      gru_w_hh.astype(bf16), gru_b_ih.reshape(1, 384),
      gru_b_hh.reshape(1, 384), rout_w, rout_b.reshape(1, 8), ew1, eb1,
      ew2, exp_b2, sel)
    return (q, p, h)


# speedup vs baseline: 1.7078x; 1.1360x over previous
import jax
import jax.numpy as jnp
from jax.experimental import pallas as pl
from jax.experimental.pallas import tpu as pltpu

_BT1 = 32
_BT2 = 32
bf16 = jnp.bfloat16


def _conv1_body(x1_r, w1_r, b1_r, y1_r):
    bt = x1_r.shape[0]
    f32 = jnp.float32
    R1 = bt * 384
    x1f = x1_r[...].reshape(R1, 128)
    M = R1 - 24
    m16 = M + 16
    accA = jnp.dot(x1f[0:m16], w1_r[...], preferred_element_type=f32)
    xcB = jnp.concatenate([x1f[0:m16, 64:128], x1f[1:m16 + 1, 0:64]],
                          axis=1)
    accB = jnp.dot(xcB, w1_r[...], preferred_element_type=f32)
    b1 = b1_r[...]
    y1_r[0:M, 0:32] = jnp.maximum(
        accA[0:M, 0:32] + accA[16:M + 16, 32:64] + b1, 0.0).astype(bf16)
    y1_r[0:M, 32:64] = jnp.maximum(
        accB[0:M, 0:32] + accB[16:M + 16, 32:64] + b1, 0.0).astype(bf16)


def _conv23_body(y1_r, w2_r, b2_r, w3_r, b3_r, y3_r, x2_s, y2_s, y3_s):
    bt = y1_r.shape[0]
    f32 = jnp.float32
    R2 = bt * 192
    y1q = y1_r[...]
    for a in range(2):
        va = y1q[:, 0:20].reshape(bt, 10, 2, 16, 64)[:, :, a]
        x2_s[:, 0:10, :, 64 * a:64 * a + 64] = va
    x2f = x2_s[...].reshape(R2, 128)
    M2 = R2 - 32
    m2b = M2 + 16
    xc2 = jnp.concatenate([x2f[0:m2b], x2f[1:m2b + 1]], axis=1)
    acc2 = jnp.dot(xc2, w2_r[...], preferred_element_type=f32)
    y2 = jnp.maximum(
        acc2[0:M2, 0:64] + acc2[16:M2 + 16, 64:128] + b2_r[...], 0.0)
    y2_s[0:M2, :] = y2.astype(bf16)

    y2f = y2_s[...]
    M3 = R2 - 48
    m3b = M3 + 32
    xc3 = jnp.concatenate([y2f[0:m3b], y2f[1:m3b + 1], y2f[2:m3b + 2]],
                          axis=1)
    acc3 = jnp.dot(xc3, w3_r[...], preferred_element_type=f32)
    y3 = jnp.maximum(acc3[0:M3, 0:64] + acc3[16:M3 + 16, 64:128]
                     + acc3[32:M3 + 32, 128:192] + b3_r[...], 0.0)
    y3_s[0:M3, :] = y3.astype(bf16)
    y3_r[...] = y3_s[...].reshape(bt, 12, 16, 64)[:, 0:7, 0:8, :]


def _head_body(y3_r, hid_r, fcw_r, fcb_r, wih_r, whh_r, bih_r, bhh_r,
               rw_r, rb_r, ew1_r, eb1_r, ew2_r, eb2_r, sel_r,
               q_r, p_r, h_r):
    f32 = jnp.float32
    hid = hid_r[...]

    feats = jnp.dot(y3_r[...], fcw_r[...], preferred_element_type=f32)
    feats = jnp.maximum(feats + fcb_r[...], 0.0)
    featsb = feats.astype(bf16)

    gi = jnp.dot(featsb, wih_r[...], preferred_element_type=f32) + bih_r[...]
    gh = jnp.dot(hid.astype(bf16), whh_r[...],
                 preferred_element_type=f32) + bhh_r[...]
    r = jax.nn.sigmoid(gi[:, 0:128] + gh[:, 0:128])
    z = jax.nn.sigmoid(gi[:, 128:256] + gh[:, 128:256])
    n = jnp.tanh(gi[:, 256:384] + r * gh[:, 256:384])
    h_new = (1.0 - z) * n + z * hid

    logits = jnp.dot(h_new, rw_r[...], preferred_element_type=f32) + rb_r[...]
    m = jnp.max(logits, axis=-1, keepdims=True)
    e = jnp.exp(logits - m)
    p = e / jnp.sum(e, axis=-1, keepdims=True)

    h1 = jnp.dot(featsb, ew1_r[...], preferred_element_type=f32) + eb1_r[...]
    h1 = jnp.maximum(h1, 0.0)
    pe = jnp.dot(p, sel_r[...], preferred_element_type=f32)
    q = jnp.dot((h1 * pe).astype(bf16), ew2_r[...],
                preferred_element_type=f32)
    q = q + jnp.dot(p, eb2_r[...], preferred_element_type=f32)

    q_r[...] = q
    p_r[...] = p
    h_r[...] = h_new


def kernel(obs, hidden, conv1_w, conv1_b, conv2_w, conv2_b, conv3_w,
           conv3_b, fc_w, fc_b, gru_w_ih, gru_w_hh, gru_b_ih, gru_b_hh,
           rout_w, rout_b, exp_w1, exp_b1, exp_w2, exp_b2):
    B = obs.shape[0]
    assert B % _BT1 == 0 and B % _BT2 == 0

    op = jnp.pad(obs.astype(bf16), ((0, 0), (0, 0), (0, 0), (0, 4)))
    xt = op.reshape(B, 4, 21, 4, 11, 2, 4)
    xt = xt.transpose(0, 2, 4, 5, 3, 1, 6).reshape(B, 21, 11, 128)
    x1 = jnp.zeros((B, 24, 16, 128), bf16).at[:, 0:21, 0:11, :].set(xt)
    x1 = x1.reshape(B, 384, 128)
    w1 = conv1_w.reshape(32, 4, 2, 4, 2, 4).transpose(4, 3, 1, 5, 2, 0)
    w1 = w1.reshape(128, 64).astype(bf16)
    w2 = conv2_w.reshape(64, 32, 2, 2, 2, 2).transpose(4, 3, 5, 1, 2, 0)
    w2 = w2.reshape(256, 128).astype(bf16)
    w3 = conv3_w.transpose(3, 1, 2, 0).reshape(192, 192).astype(bf16)
    fcw = fc_w.reshape(64, 7, 7, 512).transpose(1, 2, 0, 3)
    fcw = jnp.pad(fcw, ((0, 0), (0, 1), (0, 0), (0, 0)))
    fcw = fcw.reshape(3584, 512).astype(bf16)
    ew1 = exp_w1.transpose(1, 0, 2).reshape(512, 2048).astype(bf16)
    eb1 = exp_b1.reshape(1, 2048)
    ew2 = exp_w2.reshape(2048, 18).astype(bf16)
    sel = jnp.repeat(jnp.eye(8, dtype=jnp.float32), 256, axis=1)

    wspec2 = lambda a, b: pl.BlockSpec((a, b), lambda i: (0, 0))

    y1 = pl.pallas_call(
        _conv1_body,
        grid=(B // _BT1,),
        in_specs=[
            pl.BlockSpec((_BT1, 384, 128), lambda i: (i, 0, 0)),
            wspec2(128, 64),
            wspec2(1, 32),
        ],
        out_specs=pl.BlockSpec((_BT1 * 384, 64), lambda i: (i, 0)),
        out_shape=jax.ShapeDtypeStruct((B * 384, 64), bf16),
        compiler_params=pltpu.CompilerParams(
            vmem_limit_bytes=60 * 1024 * 1024),
    )(x1, w1, conv1_b.reshape(1, 32))

    y1g = y1.reshape(B, 24, 16, 64)

    y3 = pl.pallas_call(
        _conv23_body,
        grid=(B // _BT2,),
        in_specs=[
            pl.BlockSpec((_BT2, 24, 16, 64), lambda i: (i, 0, 0, 0)),
            wspec2(256, 128),
            wspec2(1, 64),
            wspec2(192, 192),
            wspec2(1, 64),
        ],
        out_specs=pl.BlockSpec((_BT2, 7, 8, 64), lambda i: (i, 0, 0, 0)),
        out_shape=jax.ShapeDtypeStruct((B, 7, 8, 64), bf16),
        scratch_shapes=[pltpu.VMEM((_BT2, 12, 16, 128), bf16),
                        pltpu.VMEM((_BT2 * 192, 64), bf16),
                        pltpu.VMEM((_BT2 * 192, 64), bf16)],
        compiler_params=pltpu.CompilerParams(
            vmem_limit_bytes=60 * 1024 * 1024),
    )(y1g, w2, conv2_b.reshape(1, 64), w3, conv3_b.reshape(1, 64))

    y3q = y3.reshape(B, 3584)

    q, p, h = pl.pallas_call(
        _head_body,
        grid=(1,),
        in_specs=[
            pl.BlockSpec((B, 3584), lambda i: (0, 0)),
            pl.BlockSpec((B, 128), lambda i: (0, 0)),
            wspec2(3584, 512),
            wspec2(1, 512),
            wspec2(512, 384),
            wspec2(128, 384),
            wspec2(1, 384),
            wspec2(1, 384),
            wspec2(128, 8),
            wspec2(1, 8),
            wspec2(512, 2048),
            wspec2(1, 2048),
            wspec2(2048, 18),
            wspec2(8, 18),
            wspec2(8, 2048),
        ],
        out_specs=(
            pl.BlockSpec((B, 18), lambda i: (0, 0)),
            pl.BlockSpec((B, 8), lambda i: (0, 0)),
            pl.BlockSpec((B, 128), lambda i: (0, 0)),
        ),
        out_shape=(
            jax.ShapeDtypeStruct((B, 18), jnp.float32),
            jax.ShapeDtypeStruct((B, 8), jnp.float32),
            jax.ShapeDtypeStruct((B, 128), jnp.float32),
        ),
        compiler_params=pltpu.CompilerParams(
            vmem_limit_bytes=60 * 1024 * 1024),
    )(y3q, hidden, fcw, fc_b.reshape(1, 512), gru_w_ih.astype(bf16),
      gru_w_hh.astype(bf16), gru_b_ih.reshape(1, 384),
      gru_b_hh.reshape(1, 384), rout_w, rout_b.reshape(1, 8), ew1, eb1,
      ew2, exp_b2, sel)
    return (q, p, h)
